# baseline (device time: 40143 ns/iter reference)
import jax
import jax.numpy as jnp
from jax import lax
from jax.experimental import pallas as pl
from jax.experimental.pallas import tpu as pltpu

N_DEV = 8


def kernel(x, W1, W2):
    m, _ = x.shape
    _, n = W2.shape

    def body(x_ref, w1_ref, w2_ref, out_ref, comm_ref, send_sems, recv_sems):
        my = lax.axis_index("i")
        left = lax.rem(my - 1 + N_DEV, N_DEV)
        right = lax.rem(my + 1, N_DEV)

        barrier_sem = pltpu.get_barrier_semaphore()
        for nbr in (left, right):
            pl.semaphore_signal(
                barrier_sem, inc=1,
                device_id=(nbr,), device_id_type=pl.DeviceIdType.MESH,
            )
        pl.semaphore_wait(barrier_sem, 2)

        h = jnp.maximum(
            jnp.dot(x_ref[...], w1_ref[...], preferred_element_type=jnp.float32),
            0.0,
        )
        partial = jnp.dot(h, w2_ref[...], preferred_element_type=jnp.float32)

        comm_ref[0, :, :] = partial
        acc = partial
        for hop in range(N_DEV - 1):
            rdma = pltpu.make_async_remote_copy(
                src_ref=comm_ref.at[hop],
                dst_ref=comm_ref.at[hop + 1],
                send_sem=send_sems.at[hop],
                recv_sem=recv_sems.at[hop],
                device_id=(right,),
                device_id_type=pl.DeviceIdType.MESH,
            )
            rdma.start()
            rdma.wait()
            acc = acc + comm_ref[hop + 1, :, :]
        out_ref[...] = acc

    return pl.pallas_call(
        body,
        out_shape=jax.ShapeDtypeStruct((m, n), jnp.float32),
        in_specs=[pl.BlockSpec(memory_space=pltpu.VMEM)] * 3,
        out_specs=pl.BlockSpec(memory_space=pltpu.VMEM),
        scratch_shapes=[
            pltpu.VMEM((N_DEV, m, n), jnp.float32),
            pltpu.SemaphoreType.DMA((N_DEV - 1,)),
            pltpu.SemaphoreType.DMA((N_DEV - 1,)),
        ],
        compiler_params=pltpu.CompilerParams(collective_id=0),
    )(x, W1, W2)


# device time: 16773 ns/iter; 2.3933x vs baseline; 2.3933x over previous
import jax
import jax.numpy as jnp
from jax import lax
from jax.experimental import pallas as pl
from jax.experimental.pallas import tpu as pltpu

N_DEV = 8


def kernel(x, W1, W2):
    m, _ = x.shape
    _, n = W2.shape
    mc = m // N_DEV

    def body(x_ref, w1_ref, w2_ref, out_ref, pbuf, rsbuf, redbuf, agbuf,
             rs_send, rs_recv, ag_send, ag_recv):
        my = lax.axis_index("i")

        barrier_sem = pltpu.get_barrier_semaphore()
        for d in range(1, N_DEV):
            pl.semaphore_signal(
                barrier_sem, inc=1,
                device_id=(lax.rem(my + d, N_DEV),),
                device_id_type=pl.DeviceIdType.MESH,
            )
        pl.semaphore_wait(barrier_sem, N_DEV - 1)

        h = jnp.maximum(
            jnp.dot(x_ref[...], w1_ref[...], preferred_element_type=jnp.float32),
            0.0,
        )
        partial = jnp.dot(h, w2_ref[...], preferred_element_type=jnp.float32)
        pbuf[...] = partial

        rs_descs = []
        for d in range(1, N_DEV):
            t = lax.rem(my + d, N_DEV)
            rdma = pltpu.make_async_remote_copy(
                src_ref=pbuf.at[pl.ds(t * mc, mc), :],
                dst_ref=rsbuf.at[pl.ds(my * mc, mc), :],
                send_sem=rs_send.at[d - 1],
                recv_sem=rs_recv.at[d - 1],
                device_id=(t,),
                device_id_type=pl.DeviceIdType.MESH,
            )
            rdma.start()
            rs_descs.append(rdma)

        rsbuf[pl.ds(my * mc, mc), :] = pbuf[pl.ds(my * mc, mc), :]

        for d in range(1, N_DEV):
            s = lax.rem(my + d, N_DEV)
            recv = pltpu.make_async_remote_copy(
                src_ref=pbuf.at[pl.ds(0, mc), :],
                dst_ref=rsbuf.at[pl.ds(s * mc, mc), :],
                send_sem=rs_send.at[0],
                recv_sem=rs_recv.at[N_DEV - d - 1],
                device_id=(s,),
                device_id_type=pl.DeviceIdType.MESH,
            )
            recv.wait_recv()

        red = jnp.sum(rsbuf[...].reshape(N_DEV, mc, n), axis=0)
        redbuf[...] = red

        ag_descs = []
        for d in range(1, N_DEV):
            t = lax.rem(my + d, N_DEV)
            rdma = pltpu.make_async_remote_copy(
                src_ref=redbuf,
                dst_ref=agbuf.at[pl.ds(my * mc, mc), :],
                send_sem=ag_send.at[d - 1],
                recv_sem=ag_recv.at[d - 1],
                device_id=(t,),
                device_id_type=pl.DeviceIdType.MESH,
            )
            rdma.start()
            ag_descs.append(rdma)

        agbuf[pl.ds(my * mc, mc), :] = red

        for d in range(1, N_DEV):
            s = lax.rem(my + d, N_DEV)
            recv = pltpu.make_async_remote_copy(
                src_ref=redbuf,
                dst_ref=agbuf.at[pl.ds(s * mc, mc), :],
                send_sem=ag_send.at[0],
                recv_sem=ag_recv.at[N_DEV - d - 1],
                device_id=(s,),
                device_id_type=pl.DeviceIdType.MESH,
            )
            recv.wait_recv()

        out_ref[...] = agbuf[...]

        for rdma in rs_descs:
            rdma.wait_send()
        for rdma in ag_descs:
            rdma.wait_send()

    return pl.pallas_call(
        body,
        out_shape=jax.ShapeDtypeStruct((m, n), jnp.float32),
        in_specs=[pl.BlockSpec(memory_space=pltpu.VMEM)] * 3,
        out_specs=pl.BlockSpec(memory_space=pltpu.VMEM),
        scratch_shapes=[
            pltpu.VMEM((m, n), jnp.float32),
            pltpu.VMEM((m, n), jnp.float32),
            pltpu.VMEM((mc, n), jnp.float32),
            pltpu.VMEM((m, n), jnp.float32),
            pltpu.SemaphoreType.DMA((N_DEV - 1,)),
            pltpu.SemaphoreType.DMA((N_DEV - 1,)),
            pltpu.SemaphoreType.DMA((N_DEV - 1,)),
            pltpu.SemaphoreType.DMA((N_DEV - 1,)),
        ],
        compiler_params=pltpu.CompilerParams(collective_id=0),
    )(x, W1, W2)


# device time: 16310 ns/iter; 2.4613x vs baseline; 1.0284x over previous
import jax
import jax.numpy as jnp
from jax import lax
from jax.experimental import pallas as pl
from jax.experimental.pallas import tpu as pltpu

N_DEV = 8


def kernel(x, W1, W2):
    m, _ = x.shape
    _, n = W2.shape
    mc = m // N_DEV

    def body(x_ref, w1_ref, w2_ref, out_ref, pbuf, rsbuf, redbuf, agbuf,
             rs_send, rs_recv, ag_send, ag_recv):
        my = lax.axis_index("i")

        barrier_sem = pltpu.get_barrier_semaphore()
        for d in range(1, N_DEV):
            pl.semaphore_signal(
                barrier_sem, inc=1,
                device_id=(lax.rem(my + d, N_DEV),),
                device_id_type=pl.DeviceIdType.MESH,
            )

        h = jnp.maximum(
            jnp.dot(x_ref[...], w1_ref[...], preferred_element_type=jnp.float32),
            0.0,
        )
        partial = jnp.dot(h, w2_ref[...], preferred_element_type=jnp.float32)
        pbuf[...] = partial

        pl.semaphore_wait(barrier_sem, N_DEV - 1)

        rs_descs = []
        for d in range(1, N_DEV):
            t = lax.rem(my + d, N_DEV)
            rdma = pltpu.make_async_remote_copy(
                src_ref=pbuf.at[pl.ds(t * mc, mc), :],
                dst_ref=rsbuf.at[pl.ds(my * mc, mc), :],
                send_sem=rs_send.at[d - 1],
                recv_sem=rs_recv.at[d - 1],
                device_id=(t,),
                device_id_type=pl.DeviceIdType.MESH,
            )
            rdma.start()
            rs_descs.append(rdma)

        rsbuf[pl.ds(my * mc, mc), :] = pbuf[pl.ds(my * mc, mc), :]

        for d in range(1, N_DEV):
            s = lax.rem(my + d, N_DEV)
            recv = pltpu.make_async_remote_copy(
                src_ref=pbuf.at[pl.ds(0, mc), :],
                dst_ref=rsbuf.at[pl.ds(s * mc, mc), :],
                send_sem=rs_send.at[0],
                recv_sem=rs_recv.at[N_DEV - d - 1],
                device_id=(s,),
                device_id_type=pl.DeviceIdType.MESH,
            )
            recv.wait_recv()

        red = jnp.sum(rsbuf[...].reshape(N_DEV, mc, n), axis=0)
        redbuf[...] = red

        ag_descs = []
        for d in range(1, N_DEV):
            t = lax.rem(my + d, N_DEV)
            rdma = pltpu.make_async_remote_copy(
                src_ref=redbuf,
                dst_ref=agbuf.at[pl.ds(my * mc, mc), :],
                send_sem=ag_send.at[d - 1],
                recv_sem=ag_recv.at[d - 1],
                device_id=(t,),
                device_id_type=pl.DeviceIdType.MESH,
            )
            rdma.start()
            ag_descs.append(rdma)

        agbuf[pl.ds(my * mc, mc), :] = red

        for d in range(1, N_DEV):
            s = lax.rem(my + d, N_DEV)
            recv = pltpu.make_async_remote_copy(
                src_ref=redbuf,
                dst_ref=agbuf.at[pl.ds(s * mc, mc), :],
                send_sem=ag_send.at[0],
                recv_sem=ag_recv.at[N_DEV - d - 1],
                device_id=(s,),
                device_id_type=pl.DeviceIdType.MESH,
            )
            recv.wait_recv()

        out_ref[...] = agbuf[...]

        for rdma in rs_descs:
            rdma.wait_send()
        for rdma in ag_descs:
            rdma.wait_send()

    return pl.pallas_call(
        body,
        out_shape=jax.ShapeDtypeStruct((m, n), jnp.float32),
        in_specs=[pl.BlockSpec(memory_space=pltpu.VMEM)] * 3,
        out_specs=pl.BlockSpec(memory_space=pltpu.VMEM),
        scratch_shapes=[
            pltpu.VMEM((m, n), jnp.float32),
            pltpu.VMEM((m, n), jnp.float32),
            pltpu.VMEM((mc, n), jnp.float32),
            pltpu.VMEM((m, n), jnp.float32),
            pltpu.SemaphoreType.DMA((N_DEV - 1,)),
            pltpu.SemaphoreType.DMA((N_DEV - 1,)),
            pltpu.SemaphoreType.DMA((N_DEV - 1,)),
            pltpu.SemaphoreType.DMA((N_DEV - 1,)),
        ],
        compiler_params=pltpu.CompilerParams(collective_id=0),
    )(x, W1, W2)


# device time: 14522 ns/iter; 2.7643x vs baseline; 1.1231x over previous
import jax
import jax.numpy as jnp
from jax import lax
from jax.experimental import pallas as pl
from jax.experimental.pallas import tpu as pltpu

N_DEV = 8


def kernel(x, W1, W2):
    m, _ = x.shape
    _, n = W2.shape
    mc = m // N_DEV

    def body(x_ref, w1_ref, w2_ref, out_ref, pbuf, rsbuf, redbuf,
             rs_send, rs_recv, ag_send, ag_recv):
        my = lax.axis_index("i")

        barrier_sem = pltpu.get_barrier_semaphore()
        for d in range(1, N_DEV):
            pl.semaphore_signal(
                barrier_sem, inc=1,
                device_id=(lax.rem(my + d, N_DEV),),
                device_id_type=pl.DeviceIdType.MESH,
            )

        h = jnp.maximum(
            jnp.dot(x_ref[...], w1_ref[...], preferred_element_type=jnp.float32),
            0.0,
        )
        partial = jnp.dot(h, w2_ref[...], preferred_element_type=jnp.float32)
        pbuf[...] = partial

        pl.semaphore_wait(barrier_sem, N_DEV - 1)

        rs_descs = []
        for d in range(1, N_DEV):
            t = lax.rem(my + d, N_DEV)
            rdma = pltpu.make_async_remote_copy(
                src_ref=pbuf.at[pl.ds(t * mc, mc), :],
                dst_ref=rsbuf.at[pl.ds(my * mc, mc), :],
                send_sem=rs_send.at[d - 1],
                recv_sem=rs_recv.at[d - 1],
                device_id=(t,),
                device_id_type=pl.DeviceIdType.MESH,
            )
            rdma.start()
            rs_descs.append(rdma)

        rsbuf[pl.ds(my * mc, mc), :] = pbuf[pl.ds(my * mc, mc), :]

        red = rsbuf[pl.ds(my * mc, mc), :]
        for d in range(1, N_DEV):
            s = lax.rem(my + d, N_DEV)
            recv = pltpu.make_async_remote_copy(
                src_ref=pbuf.at[pl.ds(0, mc), :],
                dst_ref=rsbuf.at[pl.ds(s * mc, mc), :],
                send_sem=rs_send.at[0],
                recv_sem=rs_recv.at[N_DEV - d - 1],
                device_id=(s,),
                device_id_type=pl.DeviceIdType.MESH,
            )
            recv.wait_recv()
            red = red + rsbuf[pl.ds(s * mc, mc), :]
        redbuf[...] = red

        ag_descs = []
        for d in range(1, N_DEV):
            t = lax.rem(my + d, N_DEV)
            rdma = pltpu.make_async_remote_copy(
                src_ref=redbuf,
                dst_ref=out_ref.at[pl.ds(my * mc, mc), :],
                send_sem=ag_send.at[d - 1],
                recv_sem=ag_recv.at[d - 1],
                device_id=(t,),
                device_id_type=pl.DeviceIdType.MESH,
            )
            rdma.start()
            ag_descs.append(rdma)

        out_ref[pl.ds(my * mc, mc), :] = red

        for d in range(1, N_DEV):
            s = lax.rem(my + d, N_DEV)
            recv = pltpu.make_async_remote_copy(
                src_ref=redbuf,
                dst_ref=out_ref.at[pl.ds(s * mc, mc), :],
                send_sem=ag_send.at[0],
                recv_sem=ag_recv.at[N_DEV - d - 1],
                device_id=(s,),
                device_id_type=pl.DeviceIdType.MESH,
            )
            recv.wait_recv()

        for rdma in rs_descs:
            rdma.wait_send()
        for rdma in ag_descs:
            rdma.wait_send()

    return pl.pallas_call(
        body,
        out_shape=jax.ShapeDtypeStruct((m, n), jnp.float32),
        in_specs=[pl.BlockSpec(memory_space=pltpu.VMEM)] * 3,
        out_specs=pl.BlockSpec(memory_space=pltpu.VMEM),
        scratch_shapes=[
            pltpu.VMEM((m, n), jnp.float32),
            pltpu.VMEM((m, n), jnp.float32),
            pltpu.VMEM((mc, n), jnp.float32),
            pltpu.SemaphoreType.DMA((N_DEV - 1,)),
            pltpu.SemaphoreType.DMA((N_DEV - 1,)),
            pltpu.SemaphoreType.DMA((N_DEV - 1,)),
            pltpu.SemaphoreType.DMA((N_DEV - 1,)),
        ],
        compiler_params=pltpu.CompilerParams(collective_id=0),
    )(x, W1, W2)
